# baseline (speedup 1.0000x reference)
import functools

import jax
import jax.numpy as jnp
from jax import lax
from jax.experimental import pallas as pl
from jax.experimental.pallas import tpu as pltpu
from jax.experimental.pallas import tpu_sc as plsc

N_NODES = 10000
F = 128
N_EDGES = 320000
NC, NS = 2, 16
CHUNK = 64
PER_TILE = N_EDGES // (NC * NS)
NCHUNK = PER_TILE // CHUNK
TAIL = PER_TILE - NCHUNK * CHUNK
ACC_ROWS = N_NODES
ZROWS = 625
OUT_ROWS_MAIN = 632
OUT_ROWS_LAST = N_NODES - (NS - 1) * OUT_ROWS_MAIN


def _sc_aggregate(x, src, dst):
    mesh = plsc.VectorSubcoreMesh(core_axis_name="c", subcore_axis_name="s")

    @functools.partial(
        pl.kernel,
        out_type=jax.ShapeDtypeStruct((NC, N_NODES, F), jnp.float32),
        mesh=mesh,
        scratch_types=[
            pltpu.VMEM((PER_TILE,), jnp.int32),
            pltpu.VMEM((CHUNK,), jnp.int32),
            pltpu.VMEM((CHUNK,), jnp.int32),
            pltpu.VMEM((CHUNK,), jnp.int32),
            pltpu.VMEM((CHUNK,), jnp.int32),
            pltpu.VMEM((TAIL,), jnp.int32),
            pltpu.VMEM((CHUNK, F), jnp.float32),
            pltpu.VMEM((CHUNK, F), jnp.float32),
            pltpu.VMEM((CHUNK, F), jnp.float32),
            pltpu.VMEM((CHUNK, F), jnp.float32),
            pltpu.VMEM_SHARED((ACC_ROWS, F), jnp.float32),
            pltpu.SemaphoreType.DMA,
            pltpu.SemaphoreType.DMA,
            pltpu.SemaphoreType.DMA,
            pltpu.SemaphoreType.DMA,
            pltpu.SemaphoreType.DMA,
            pltpu.SemaphoreType.DMA,
            pltpu.SemaphoreType.DMA,
            pltpu.SemaphoreType.DMA,
            pltpu.SemaphoreType.DMA,
            pltpu.SemaphoreType.DMA,
            pltpu.SemaphoreType.DMA,
            pltpu.SemaphoreType.DMA,
        ],
    )
    def k(x_hbm, src_hbm, dst_hbm, out_hbm, src_v,
          dst_0, dst_1, dst_2, dst_3, dst_t,
          rows_0, rows_1, rows_2, rows_3, acc,
          sem_g0, sem_g1, sem_g2, sem_g3, sem_d0, sem_d1, sem_d2, sem_d3,
          sem_s0, sem_s1, sem_s2, sem_s3):
        dst_bufs = (dst_0, dst_1, dst_2, dst_3)
        row_bufs = (rows_0, rows_1, rows_2, rows_3)
        g_sems = (sem_g0, sem_g1, sem_g2, sem_g3)
        d_sems = (sem_d0, sem_d1, sem_d2, sem_d3)
        s_sems = (sem_s0, sem_s1, sem_s2, sem_s3)
        c = lax.axis_index("c")
        s = lax.axis_index("s")
        base = (c * NS + s) * PER_TILE

        src_load = pltpu.make_async_copy(src_hbm.at[pl.ds(base, PER_TILE)],
                                         src_v, sem_d0)
        src_load.start()

        @pl.loop(0, CHUNK)
        def _(i):
            for k16 in range(0, F, 16):
                rows_0[i, pl.ds(k16, 16)] = jnp.zeros((16,), jnp.float32)

        zbase = s * ZROWS
        zcopies = []
        for r in range(ZROWS // CHUNK):
            zcopies.append(pltpu.make_async_copy(
                rows_0, acc.at[pl.ds(zbase + r * CHUNK, CHUNK)], sem_s0))
        zrem = ZROWS % CHUNK
        zcopies.append(pltpu.make_async_copy(
            rows_0.at[pl.ds(0, zrem)],
            acc.at[pl.ds(zbase + ZROWS - zrem, zrem)], sem_s0))
        for cp in zcopies:
            cp.start()
        for cp in zcopies:
            cp.wait()
        src_load.wait()

        plsc.subcore_barrier()

        def start_fetch(j, b):
            pltpu.make_async_copy(dst_hbm.at[pl.ds(base + j * CHUNK, CHUNK)],
                                  dst_bufs[b], d_sems[b]).start()
            pltpu.make_async_copy(x_hbm.at[src_v.at[pl.ds(j * CHUNK, CHUNK)]],
                                  row_bufs[b], g_sems[b]).start()

        def wait_fetch(j, b):
            pltpu.make_async_copy(dst_hbm.at[pl.ds(base + j * CHUNK, CHUNK)],
                                  dst_bufs[b], d_sems[b]).wait()
            pltpu.make_async_copy(x_hbm.at[src_v.at[pl.ds(j * CHUNK, CHUNK)]],
                                  row_bufs[b], g_sems[b]).wait()

        def start_scatter(b):
            pltpu.make_async_copy(row_bufs[b], acc.at[dst_bufs[b]],
                                  s_sems[b]).start(add=True)

        def wait_scatter(b):
            pltpu.make_async_copy(row_bufs[b], acc.at[dst_bufs[b]],
                                  s_sems[b]).wait()

        NBUF, LA = 4, 2

        @pl.loop(0, NCHUNK + NBUF, step=NBUF)
        def _(jj):
            for db in range(NBUF):
                i = jj + db
                bp = (db - LA) % NBUF

                @pl.when(i < NCHUNK)
                def _():
                    @pl.when(i >= NBUF)
                    def _():
                        wait_scatter(db)
                    start_fetch(i, db)

                p = i - LA

                @pl.when(jnp.logical_and(p >= 0, p < NCHUNK))
                def _():
                    wait_fetch(p, bp)
                    start_scatter(bp)

        for b in range(NBUF):
            wait_scatter(b)

        toff = NCHUNK * CHUNK
        pltpu.sync_copy(dst_hbm.at[pl.ds(base + toff, TAIL)], dst_t)
        pltpu.async_copy(x_hbm.at[src_v.at[pl.ds(toff, TAIL)]],
                         rows_0.at[pl.ds(0, TAIL)], sem_g0).wait()
        pltpu.sync_copy(rows_0.at[pl.ds(0, TAIL)], acc.at[dst_t], add=True)

        plsc.subcore_barrier()

        @pl.when(s < NS - 1)
        def _():
            pltpu.sync_copy(
                acc.at[pl.ds(s * OUT_ROWS_MAIN, OUT_ROWS_MAIN)],
                out_hbm.at[c, pl.ds(s * OUT_ROWS_MAIN, OUT_ROWS_MAIN)])

        @pl.when(s == NS - 1)
        def _():
            pltpu.sync_copy(
                acc.at[pl.ds((NS - 1) * OUT_ROWS_MAIN, OUT_ROWS_LAST)],
                out_hbm.at[c, pl.ds((NS - 1) * OUT_ROWS_MAIN, OUT_ROWS_LAST)])

    return k(x, src, dst)


BLK = 2000


def _tc_body(p_ref, w_ref, b_ref, o_ref):
    ssum = p_ref[0] + p_ref[1]
    o_ref[...] = lax.dot_general(
        ssum, w_ref[...], (((1,), (0,)), ((), ())),
        preferred_element_type=jnp.float32,
        precision=lax.Precision.HIGHEST) + b_ref[...]


def _tc_finish(partial, W, b):
    return pl.pallas_call(
        _tc_body,
        grid=(N_NODES // BLK,),
        in_specs=[
            pl.BlockSpec((NC, BLK, F), lambda i: (0, i, 0)),
            pl.BlockSpec((F, F), lambda i: (0, 0)),
            pl.BlockSpec((1, F), lambda i: (0, 0)),
        ],
        out_specs=pl.BlockSpec((BLK, F), lambda i: (i, 0)),
        out_shape=jax.ShapeDtypeStruct((N_NODES, F), jnp.float32),
    )(partial, W, b.reshape(1, F))


def kernel(x, edge_index, W, b):
    partial = _sc_aggregate(x, edge_index[0], edge_index[1])
    return _tc_finish(partial, W, b)


# speedup vs baseline: 1.0633x; 1.0633x over previous
import functools

import jax
import jax.numpy as jnp
import numpy as np
from jax import lax
from jax.experimental import pallas as pl
from jax.experimental.pallas import tpu as pltpu
from jax.experimental.pallas import tpu_sc as plsc

N_NODES = 10000
F = 128
N_EDGES = 320000
NC, NS = 2, 16
CHUNK = 128
NCHUNK = 80
PER_TILE = CHUNK * NCHUNK
E_PAD = PER_TILE * NC * NS
ACC_ROWS = 10240
ZROWS = ACC_ROWS // NS
OUT_ROWS_MAIN = 632
OUT_ROWS_LAST = N_NODES - (NS - 1) * OUT_ROWS_MAIN

_PAD = E_PAD - N_EDGES
_PAD_EDGES = np.stack([
    np.arange(_PAD, dtype=np.int32) % N_NODES,
    (N_NODES + np.arange(_PAD) % (ACC_ROWS - N_NODES)).astype(np.int32),
])


def _sc_aggregate(x, edge_index, pad_edges):
    mesh = plsc.VectorSubcoreMesh(core_axis_name="c", subcore_axis_name="s")

    @functools.partial(
        pl.kernel,
        out_type=jax.ShapeDtypeStruct((NC, N_NODES, F), jnp.float32),
        mesh=mesh,
        scratch_types=[
            pltpu.VMEM((2, CHUNK), jnp.int32),
            pltpu.VMEM((2, CHUNK), jnp.int32),
            pltpu.VMEM((2, CHUNK), jnp.int32),
            pltpu.VMEM((2, CHUNK), jnp.int32),
            pltpu.VMEM((CHUNK, F), jnp.float32),
            pltpu.VMEM((CHUNK, F), jnp.float32),
            pltpu.VMEM_SHARED((ACC_ROWS, F), jnp.float32),
            pltpu.SemaphoreType.DMA,
            pltpu.SemaphoreType.DMA,
            pltpu.SemaphoreType.DMA,
            pltpu.SemaphoreType.DMA,
            pltpu.SemaphoreType.DMA,
            pltpu.SemaphoreType.DMA,
            pltpu.SemaphoreType.DMA,
            pltpu.SemaphoreType.DMA,
        ],
    )
    def k(x_hbm, ei_hbm, pad_hbm, out_hbm,
          idx_0, idx_1, idx_2, idx_3, rows_0, rows_1, acc,
          sem_d0, sem_d1, sem_d2, sem_d3, sem_g0, sem_g1, sem_s0, sem_s1):
        idx_bufs = (idx_0, idx_1, idx_2, idx_3)
        row_bufs = (rows_0, rows_1)
        d_sems = (sem_d0, sem_d1, sem_d2, sem_d3)
        g_sems = (sem_g0, sem_g1)
        s_sems = (sem_s0, sem_s1)
        c = lax.axis_index("c")
        s = lax.axis_index("s")
        base = (c * NS + s) * PER_TILE

        @pl.loop(0, CHUNK)
        def _(i):
            for k16 in range(0, F, 16):
                rows_0[i, pl.ds(k16, 16)] = jnp.zeros((16,), jnp.float32)

        zbase = s * ZROWS
        zcopies = [
            pltpu.make_async_copy(
                rows_0, acc.at[pl.ds(zbase + r * CHUNK, CHUNK)], sem_s0)
            for r in range(ZROWS // CHUNK)
        ]
        for cp in zcopies:
            cp.start()
        for cp in zcopies:
            cp.wait()

        plsc.subcore_barrier()

        def start_idx(j, b):
            col = base + j * CHUNK

            @pl.when(col < N_EDGES)
            def _():
                pltpu.make_async_copy(ei_hbm.at[:, pl.ds(col, CHUNK)],
                                      idx_bufs[b], d_sems[b]).start()

            @pl.when(col >= N_EDGES)
            def _():
                pltpu.make_async_copy(
                    pad_hbm.at[:, pl.ds(col - N_EDGES, CHUNK)],
                    idx_bufs[b], d_sems[b]).start()

        def wait_idx(j, b):
            pltpu.make_async_copy(ei_hbm.at[:, pl.ds(0, CHUNK)],
                                  idx_bufs[b], d_sems[b]).wait()

        def start_gather(b2, b4):
            pltpu.make_async_copy(x_hbm.at[idx_bufs[b4].at[0]],
                                  row_bufs[b2], g_sems[b2]).start()

        def wait_gather(b2, b4):
            pltpu.make_async_copy(x_hbm.at[idx_bufs[b4].at[0]],
                                  row_bufs[b2], g_sems[b2]).wait()

        def start_scatter(b2, b4):
            pltpu.make_async_copy(row_bufs[b2], acc.at[idx_bufs[b4].at[1]],
                                  s_sems[b2]).start(add=True)

        def wait_scatter(b2, b4):
            pltpu.make_async_copy(row_bufs[b2], acc.at[idx_bufs[b4].at[1]],
                                  s_sems[b2]).wait()

        start_idx(0, 0)
        start_idx(1, 1)

        @pl.loop(0, NCHUNK + 4, step=4)
        def _(jj):
            for db in range(4):
                i = jj + db
                b2 = db % 2

                @pl.when(jnp.logical_and(i >= 2, i < NCHUNK))
                def _():
                    wait_scatter(b2, (db - 2) % 4)

                @pl.when(i + 2 < NCHUNK)
                def _():
                    start_idx(i + 2, (db + 2) % 4)

                @pl.when(i < NCHUNK)
                def _():
                    wait_idx(i, db)
                    start_gather(b2, db)

                p = i - 1
                pb2 = (db - 1) % 2
                pb4 = (db - 1) % 4

                @pl.when(jnp.logical_and(p >= 0, p < NCHUNK))
                def _():
                    wait_gather(pb2, pb4)
                    start_scatter(pb2, pb4)

        wait_scatter(0, (NCHUNK - 2) % 4)
        wait_scatter(1, (NCHUNK - 1) % 4)

        plsc.subcore_barrier()

        @pl.when(s < NS - 1)
        def _():
            pltpu.sync_copy(
                acc.at[pl.ds(s * OUT_ROWS_MAIN, OUT_ROWS_MAIN)],
                out_hbm.at[c, pl.ds(s * OUT_ROWS_MAIN, OUT_ROWS_MAIN)])

        @pl.when(s == NS - 1)
        def _():
            pltpu.sync_copy(
                acc.at[pl.ds((NS - 1) * OUT_ROWS_MAIN, OUT_ROWS_LAST)],
                out_hbm.at[c, pl.ds((NS - 1) * OUT_ROWS_MAIN, OUT_ROWS_LAST)])

    return k(x, edge_index, pad_edges)


BLK = 2000


def _tc_body(p_ref, w_ref, b_ref, o_ref):
    ssum = p_ref[0] + p_ref[1]
    o_ref[...] = lax.dot_general(
        ssum, w_ref[...], (((1,), (0,)), ((), ())),
        preferred_element_type=jnp.float32,
        precision=lax.Precision.HIGHEST) + b_ref[...]


def _tc_finish(partial, W, b):
    return pl.pallas_call(
        _tc_body,
        grid=(N_NODES // BLK,),
        in_specs=[
            pl.BlockSpec((NC, BLK, F), lambda i: (0, i, 0)),
            pl.BlockSpec((F, F), lambda i: (0, 0)),
            pl.BlockSpec((1, F), lambda i: (0, 0)),
        ],
        out_specs=pl.BlockSpec((BLK, F), lambda i: (i, 0)),
        out_shape=jax.ShapeDtypeStruct((N_NODES, F), jnp.float32),
    )(partial, W, b.reshape(1, F))


def kernel(x, edge_index, W, b):
    partial = _sc_aggregate(x, edge_index, jnp.asarray(_PAD_EDGES))
    return _tc_finish(partial, W, b)


# speedup vs baseline: 1.0754x; 1.0113x over previous
import functools

import jax
import jax.numpy as jnp
import numpy as np
from jax import lax
from jax.experimental import pallas as pl
from jax.experimental.pallas import tpu as pltpu
from jax.experimental.pallas import tpu_sc as plsc

N_NODES = 10000
F = 128
N_EDGES = 320000
NC, NS = 2, 16
CHUNK = 128
NCHUNK = 80
PER_TILE = CHUNK * NCHUNK
E_PAD = PER_TILE * NC * NS
ACC_ROWS = 10048
ZROWS = ACC_ROWS // NS
OUT_ROWS_MAIN = 632
OUT_ROWS_LAST = N_NODES - (NS - 1) * OUT_ROWS_MAIN

_PAD = E_PAD - N_EDGES
_PAD_EDGES = np.stack([
    np.arange(_PAD, dtype=np.int32) % N_NODES,
    (N_NODES + np.arange(_PAD) % (ACC_ROWS - N_NODES)).astype(np.int32),
])


def _sc_aggregate(x, edge_index, pad_edges):
    mesh = plsc.VectorSubcoreMesh(core_axis_name="c", subcore_axis_name="s")

    @functools.partial(
        pl.kernel,
        out_type=jax.ShapeDtypeStruct((NC, N_NODES, F), jnp.float32),
        mesh=mesh,
        scratch_types=[
            pltpu.VMEM((2, CHUNK), jnp.int32),
            pltpu.VMEM((2, CHUNK), jnp.int32),
            pltpu.VMEM((2, CHUNK), jnp.int32),
            pltpu.VMEM((2, CHUNK), jnp.int32),
            pltpu.VMEM((CHUNK, F), jnp.float32),
            pltpu.VMEM((CHUNK, F), jnp.float32),
            pltpu.VMEM((CHUNK, F), jnp.float32),
            pltpu.VMEM_SHARED((ACC_ROWS, F), jnp.float32),
            pltpu.SemaphoreType.DMA,
            pltpu.SemaphoreType.DMA,
            pltpu.SemaphoreType.DMA,
            pltpu.SemaphoreType.DMA,
            pltpu.SemaphoreType.DMA,
            pltpu.SemaphoreType.DMA,
            pltpu.SemaphoreType.DMA,
            pltpu.SemaphoreType.DMA,
        ],
    )
    def k(x_hbm, ei_hbm, pad_hbm, out_hbm,
          idx_0, idx_1, idx_2, idx_3, rows_0, rows_1, zbuf, acc,
          sem_d0, sem_d1, sem_d2, sem_d3, sem_g0, sem_g1, sem_s0, sem_s1):
        idx_bufs = (idx_0, idx_1, idx_2, idx_3)
        row_bufs = (rows_0, rows_1)
        d_sems = (sem_d0, sem_d1, sem_d2, sem_d3)
        g_sems = (sem_g0, sem_g1)
        s_sems = (sem_s0, sem_s1)
        c = lax.axis_index("c")
        s = lax.axis_index("s")
        base = (c * NS + s) * PER_TILE

        def start_idx(j, b):
            col = base + j * CHUNK

            @pl.when(col < N_EDGES)
            def _():
                pltpu.make_async_copy(ei_hbm.at[:, pl.ds(col, CHUNK)],
                                      idx_bufs[b], d_sems[b]).start()

            @pl.when(col >= N_EDGES)
            def _():
                pltpu.make_async_copy(
                    pad_hbm.at[:, pl.ds(col - N_EDGES, CHUNK)],
                    idx_bufs[b], d_sems[b]).start()

        def wait_idx(j, b):
            pltpu.make_async_copy(ei_hbm.at[:, pl.ds(0, CHUNK)],
                                  idx_bufs[b], d_sems[b]).wait()

        def start_gather(b2, b4):
            pltpu.make_async_copy(x_hbm.at[idx_bufs[b4].at[0]],
                                  row_bufs[b2], g_sems[b2]).start()

        def wait_gather(b2, b4):
            pltpu.make_async_copy(x_hbm.at[idx_bufs[b4].at[0]],
                                  row_bufs[b2], g_sems[b2]).wait()

        def start_scatter(b2, b4):
            pltpu.make_async_copy(row_bufs[b2], acc.at[idx_bufs[b4].at[1]],
                                  s_sems[b2]).start(add=True)

        def wait_scatter(b2, b4):
            pltpu.make_async_copy(row_bufs[b2], acc.at[idx_bufs[b4].at[1]],
                                  s_sems[b2]).wait()

        for j in range(4):
            start_idx(j, j)

        @pl.loop(0, CHUNK)
        def _(i):
            for k16 in range(0, F, 16):
                zbuf[i, pl.ds(k16, 16)] = jnp.zeros((16,), jnp.float32)

        zbase = s * ZROWS
        zcopies = [
            pltpu.make_async_copy(
                zbuf, acc.at[pl.ds(zbase + r * CHUNK, CHUNK)], sem_s0)
            for r in range(ZROWS // CHUNK)
        ]
        zrem = ZROWS % CHUNK
        zcopies.append(pltpu.make_async_copy(
            zbuf.at[pl.ds(0, zrem)],
            acc.at[pl.ds(zbase + ZROWS - zrem, zrem)], sem_s0))
        for cp in zcopies:
            cp.start()

        wait_idx(0, 0)
        start_gather(0, 0)
        wait_idx(1, 1)
        start_gather(1, 1)

        for cp in zcopies:
            cp.wait()

        plsc.subcore_barrier()

        @pl.loop(0, NCHUNK + 4, step=4)
        def _(jj):
            for db in range(4):
                i = jj + db
                b2 = db % 2

                @pl.when(jnp.logical_and(i >= 2, i < NCHUNK))
                def _():
                    wait_scatter(b2, (db - 2) % 4)

                @pl.when(jnp.logical_and(i >= 2, i + 2 < NCHUNK))
                def _():
                    start_idx(i + 2, (db + 2) % 4)

                @pl.when(jnp.logical_and(i >= 2, i < NCHUNK))
                def _():
                    wait_idx(i, db)
                    start_gather(b2, db)

                p = i - 1
                pb2 = (db - 1) % 2
                pb4 = (db - 1) % 4

                @pl.when(jnp.logical_and(p >= 0, p < NCHUNK))
                def _():
                    wait_gather(pb2, pb4)
                    start_scatter(pb2, pb4)

        wait_scatter(0, (NCHUNK - 2) % 4)
        wait_scatter(1, (NCHUNK - 1) % 4)

        plsc.subcore_barrier()

        @pl.when(s < NS - 1)
        def _():
            pltpu.sync_copy(
                acc.at[pl.ds(s * OUT_ROWS_MAIN, OUT_ROWS_MAIN)],
                out_hbm.at[c, pl.ds(s * OUT_ROWS_MAIN, OUT_ROWS_MAIN)])

        @pl.when(s == NS - 1)
        def _():
            pltpu.sync_copy(
                acc.at[pl.ds((NS - 1) * OUT_ROWS_MAIN, OUT_ROWS_LAST)],
                out_hbm.at[c, pl.ds((NS - 1) * OUT_ROWS_MAIN, OUT_ROWS_LAST)])

    return k(x, edge_index, pad_edges)


BLK = 2000


def _tc_body(p_ref, w_ref, b_ref, o_ref):
    ssum = p_ref[0] + p_ref[1]
    o_ref[...] = lax.dot_general(
        ssum, w_ref[...], (((1,), (0,)), ((), ())),
        preferred_element_type=jnp.float32,
        precision=lax.Precision.HIGHEST) + b_ref[...]


def _tc_finish(partial, W, b):
    return pl.pallas_call(
        _tc_body,
        grid=(N_NODES // BLK,),
        in_specs=[
            pl.BlockSpec((NC, BLK, F), lambda i: (0, i, 0)),
            pl.BlockSpec((F, F), lambda i: (0, 0)),
            pl.BlockSpec((1, F), lambda i: (0, 0)),
        ],
        out_specs=pl.BlockSpec((BLK, F), lambda i: (i, 0)),
        out_shape=jax.ShapeDtypeStruct((N_NODES, F), jnp.float32),
    )(partial, W, b.reshape(1, F))


def kernel(x, edge_index, W, b):
    partial = _sc_aggregate(x, edge_index, jnp.asarray(_PAD_EDGES))
    return _tc_finish(partial, W, b)
